# baseline (speedup 1.0000x reference)
import functools

import jax
import jax.numpy as jnp
from jax import lax
from jax.experimental import pallas as pl
from jax.experimental.pallas import tpu as pltpu
from jax.experimental.pallas import tpu_sc as plsc


def _leaky(x, slope=0.01):
    return jnp.where(x > 0, x, slope * x)


def _sc_geometry():
    try:
        info = plsc.get_sparse_core_info()
        return info.num_cores, info.num_subcores, info.num_lanes
    except Exception:
        return 2, 16, 16


def _make_spmm(Np, d, NC, NS, CH, C, CH0, CH1):
    mesh = plsc.VectorSubcoreMesh(core_axis_name="c", subcore_axis_name="s")
    rpt = Np // NS
    BC = min(128, C)
    nb = rpt // BC
    nk = d // 16

    R = 8
    CHM = CH

    @functools.partial(
        pl.kernel,
        out_type=jax.ShapeDtypeStruct((NC, Np, d), jnp.float32),
        mesh=mesh,
        scratch_types=(
            [pltpu.VMEM((8 * R, C), jnp.int32)] +
            [pltpu.VMEM((C, d), jnp.float32) for _ in range(R)] +
            [pltpu.VMEM_SHARED((Np, d), jnp.float32)] +
            [pltpu.SemaphoreType.DMA for _ in range(3 * R)]
        ),
    )
    def spmm(xs, ep, out, ibuf, *rest):
        gbufs = rest[:R]
        acc = rest[R]
        isems = rest[R + 1:R + 1 + R]
        gsems = rest[R + 1 + R:R + 1 + 2 * R]
        ssems = rest[R + 1 + 2 * R:R + 1 + 3 * R]
        cid = lax.axis_index("c")
        sid = lax.axis_index("s")
        wid = cid * NS + sid
        g0 = gbufs[0]
        zeros16 = jnp.zeros((16,), jnp.float32)

        def zrow(i, _):
            for k in range(nk):
                g0[i, pl.ds(k * 16, 16)] = zeros16
            return 0

        lax.fori_loop(0, BC, zrow, 0)
        r0 = sid * rpt

        def zacc(j, _):
            pltpu.sync_copy(g0.at[pl.ds(0, BC)],
                            acc.at[pl.ds(r0 + j * BC, BC)])
            return 0

        lax.fori_loop(0, nb, zacc, 0)
        plsc.subcore_barrier()

        def start_idx(ch, s):
            pltpu.async_copy(ep.at[pl.ds((wid * CHM + ch) * 8, 8)],
                             ibuf.at[pl.ds(8 * s, 8)], isems[s])

        def wait_idx(s):
            pltpu.make_async_copy(ep.at[pl.ds(0, 8)],
                                  ibuf.at[pl.ds(8 * s, 8)],
                                  isems[s]).wait()

        def start_gather(b):
            pltpu.async_copy(xs.at[ibuf.at[8 * b]], gbufs[b], gsems[b])

        def wait_gather(b):
            pltpu.make_async_copy(xs.at[ibuf.at[8 * b]], gbufs[b],
                                  gsems[b]).wait()

        def start_scatter(b):
            pltpu.async_copy(gbufs[b], acc.at[ibuf.at[8 * b + 1]], ssems[b],
                             add=True)

        def wait_scatter(b):
            pltpu.make_async_copy(gbufs[b], acc.at[ibuf.at[8 * b + 1]],
                                  ssems[b]).wait()

        def scale(b):
            gb = gbufs[b]

            def egroup(g, _):
                wv = lax.bitcast_convert_type(
                    ibuf[8 * b + 2, pl.ds(g * 16, 16)], jnp.float32)
                for j in range(16):
                    wb = jnp.broadcast_to(wv[j], (16,))
                    e = g * 16 + j
                    for k in range(nk):
                        sl = pl.ds(k * 16, 16)
                        gb[e, sl] = gb[e, sl] * wb
                return 0

            lax.fori_loop(0, C // 16, egroup, 0)

        nch = jnp.where(cid == 0, CH0, CH1)

        for s in range(R - 1):
            start_idx(s, s)
        for s in range(R - 3):
            wait_idx(s)
            start_gather(s)

        def octet(g, _):
            for boff in range(R):
                ch = R * g + boff
                b = boff
                wait_gather(b)
                scale(b)
                start_scatter(b)
                b7 = (b + R - 1) % R
                b5 = (b + R - 3) % R

                @pl.when(ch + R - 1 < nch)
                def _():
                    @pl.when(ch >= 1)
                    def _():
                        wait_scatter(b7)
                    start_idx(ch + R - 1, b7)

                @pl.when(ch + R - 3 < nch)
                def _():
                    wait_idx(b5)
                    start_gather(b5)
            return 0

        lax.fori_loop(0, nch // R, octet, 0)
        for b in range(R):
            wait_scatter(b)
        plsc.subcore_barrier()

        def wout(j, _):
            pltpu.sync_copy(acc.at[pl.ds(r0 + j * BC, BC)],
                            g0.at[pl.ds(0, BC)])
            pltpu.sync_copy(g0.at[pl.ds(0, BC)],
                            out.at[cid, pl.ds(r0 + j * BC, BC)])
            return 0

        lax.fori_loop(0, nb, wout, 0)

    return spmm


def _make_deg(Np, NC, NS, CH, C):
    mesh = plsc.VectorSubcoreMesh(core_axis_name="c", subcore_axis_name="s")
    rpt = Np // NS

    @functools.partial(
        pl.kernel,
        out_type=jax.ShapeDtypeStruct((NC, Np), jnp.float32),
        mesh=mesh,
        scratch_types=[
            pltpu.VMEM((CH, C), jnp.int32),
            pltpu.VMEM((CH, C), jnp.float32),
            pltpu.VMEM((rpt,), jnp.float32),
            pltpu.VMEM_SHARED((Np,), jnp.float32),
        ],
    )
    def deg(colp, wp, out, col_v, w_v, dbuf, acc):
        cid = lax.axis_index("c")
        sid = lax.axis_index("s")
        wid = cid * NS + sid
        pltpu.sync_copy(colp.at[wid], col_v)
        pltpu.sync_copy(wp.at[wid], w_v)

        zeros16 = jnp.zeros((16,), jnp.float32)

        def zrow(i, _):
            dbuf[pl.ds(i * 16, 16)] = zeros16
            return 0

        lax.fori_loop(0, rpt // 16, zrow, 0)
        r0 = sid * rpt
        pltpu.sync_copy(dbuf, acc.at[pl.ds(r0, rpt)])
        plsc.subcore_barrier()

        def chunk(ch, _):
            pltpu.sync_copy(w_v.at[ch], acc.at[col_v.at[ch]], add=True)
            return 0

        lax.fori_loop(0, CH, chunk, 0)
        plsc.subcore_barrier()
        pltpu.sync_copy(acc.at[pl.ds(r0, rpt)], dbuf)
        pltpu.sync_copy(dbuf, out.at[cid, pl.ds(r0, rpt)])

    return deg


def _prep_body(d0_ref, d1_ref, dg_ref, dinv_ref):
    deg = d0_ref[...] + d1_ref[...]
    dg_ref[...] = lax.rsqrt(deg + 1.0)
    dinv_ref[...] = jnp.where(deg > 0, 1.0 / deg, 0.0)


def _tc_prep(d0, d1):
    N = d0.shape[0]
    return pl.pallas_call(
        _prep_body,
        out_shape=(jax.ShapeDtypeStruct((N, 1), jnp.float32),
                   jax.ShapeDtypeStruct((N, 1), jnp.float32)),
    )(d0, d1)


def _scale_body(f_ref, s_ref, o_ref):
    o_ref[...] = f_ref[...] * s_ref[...]


def _tc_scale(f, s, BN):
    N, d = f.shape
    return pl.pallas_call(
        _scale_body,
        grid=(N // BN,),
        in_specs=[pl.BlockSpec((BN, d), lambda i: (i, 0)),
                  pl.BlockSpec((BN, 1), lambda i: (i, 0))],
        out_specs=pl.BlockSpec((BN, d), lambda i: (i, 0)),
        out_shape=jax.ShapeDtypeStruct((N, d), jnp.float32),
    )(f, s)


def _gcn_body(p0_ref, p1_ref, g_ref, dg_ref, gn_ref, br_ref):
    fn = dg_ref[...] * (p0_ref[...] + p1_ref[...] + g_ref[...])
    gn_ref[...] = dg_ref[...] * fn
    br_ref[...] = _leaky(fn)


def _tc_combine_gcn(p0, p1, g, dg, BN):
    N, d = g.shape
    return pl.pallas_call(
        _gcn_body,
        grid=(N // BN,),
        in_specs=[pl.BlockSpec((BN, d), lambda i: (i, 0)),
                  pl.BlockSpec((BN, d), lambda i: (i, 0)),
                  pl.BlockSpec((BN, d), lambda i: (i, 0)),
                  pl.BlockSpec((BN, 1), lambda i: (i, 0))],
        out_specs=(pl.BlockSpec((BN, d), lambda i: (i, 0)),
                   pl.BlockSpec((BN, d), lambda i: (i, 0))),
        out_shape=(jax.ShapeDtypeStruct((N, d), jnp.float32),
                   jax.ShapeDtypeStruct((N, d), jnp.float32)),
    )(p0, p1, g, dg)


def _sct_body(p0_ref, p1_ref, fp_ref, dinv_ref, m_ref, fn_ref, dx_ref, br_ref):
    fn = 0.5 * fp_ref[...] + 0.5 * (p0_ref[...] + p1_ref[...])
    fn_ref[...] = fn
    dx_ref[...] = dinv_ref[...] * fn
    ad = jnp.abs(fp_ref[...] - fn)
    m = m_ref[...]
    br_ref[...] = jnp.where(
        ad > 0, jnp.exp(m * jnp.log(jnp.maximum(ad, 1e-38))), 0.0)


def _tc_combine_sct(p0, p1, fp, dinv, m, BN):
    N, d = fp.shape
    return pl.pallas_call(
        _sct_body,
        grid=(N // BN,),
        in_specs=[pl.BlockSpec((BN, d), lambda i: (i, 0)),
                  pl.BlockSpec((BN, d), lambda i: (i, 0)),
                  pl.BlockSpec((BN, d), lambda i: (i, 0)),
                  pl.BlockSpec((BN, 1), lambda i: (i, 0)),
                  pl.BlockSpec((1, 1), lambda i: (0, 0))],
        out_specs=(pl.BlockSpec((BN, d), lambda i: (i, 0)),
                   pl.BlockSpec((BN, d), lambda i: (i, 0)),
                   pl.BlockSpec((BN, d), lambda i: (i, 0))),
        out_shape=(jax.ShapeDtypeStruct((N, d), jnp.float32),
                   jax.ShapeDtypeStruct((N, d), jnp.float32),
                   jax.ShapeDtypeStruct((N, d), jnp.float32)),
    )(p0, p1, fp, dinv, m)


def _attend_body(x_ref, h0, h1, h2, h3, h4, h5, a_ref, w1_ref, b1_ref,
                 w2_ref, b2_ref, o_ref):
    d = x_ref.shape[1]
    a1 = a_ref[pl.ds(0, d), :]
    a2 = a_ref[pl.ds(d, d), :]
    c = jnp.dot(jnp.maximum(x_ref[...], 0.0), a1,
                preferred_element_type=jnp.float32)
    hs = [h0[...], h1[...], h2[...], h3[...], h4[...], h5[...]]
    es = [c + jnp.dot(jnp.maximum(h, 0.0), a2,
                      preferred_element_type=jnp.float32) for h in hs]
    e = jnp.concatenate(es, axis=1)
    e = e - jnp.max(e, axis=1, keepdims=True)
    ex = jnp.exp(e)
    att = ex / jnp.sum(ex, axis=1, keepdims=True)
    hp = att[:, 0:1] * hs[0]
    for k in range(1, 6):
        hp = hp + att[:, k:k + 1] * hs[k]
    hp = hp * (1.0 / 6.0)
    t = _leaky(lax.dot_general(hp, w1_ref[...], (((1,), (1,)), ((), ())),
                               preferred_element_type=jnp.float32)
               + b1_ref[...])
    o_ref[...] = _leaky(
        lax.dot_general(t, w2_ref[...], (((1,), (1,)), ((), ())),
                        preferred_element_type=jnp.float32) + b2_ref[...])


def _tc_attend(x, hs, a, w1, b1, w2, b2, BN):
    N, d = x.shape
    blk = pl.BlockSpec((BN, d), lambda i: (i, 0))
    return pl.pallas_call(
        _attend_body,
        grid=(N // BN,),
        in_specs=[blk, blk, blk, blk, blk, blk, blk,
                  pl.BlockSpec((2 * d, 1), lambda i: (0, 0)),
                  pl.BlockSpec((d, d), lambda i: (0, 0)),
                  pl.BlockSpec((1, d), lambda i: (0, 0)),
                  pl.BlockSpec((d, d), lambda i: (0, 0)),
                  pl.BlockSpec((1, d), lambda i: (0, 0))],
        out_specs=blk,
        out_shape=jax.ShapeDtypeStruct((N, d), jnp.float32),
    )(x, *hs, a, w1, b1, w2, b2)


def kernel(X, edge_index, edge_weight, W1, b1, W2, b2, a, moment):
    N, d = X.shape
    E = edge_weight.shape[0]
    NC, NS, _ = _sc_geometry()
    T = NC * NS
    C = 32
    tot = max(16, 8 * (-(-E // (NS * C * 8))))
    FR0 = 0.63
    CH0 = max(8, 8 * round(tot * FR0 / 8))
    CH1 = tot - CH0
    CHm = max(CH0, CH1)
    pad = NS * tot * C - E
    rowf = jnp.pad(edge_index[0], (0, pad))
    colf = jnp.pad(edge_index[1], (0, pad))
    wf = jnp.pad(edge_weight, (0, pad))
    wbits = lax.bitcast_convert_type(wf, jnp.int32)

    def _slab(x):
        p0 = x[:NS * CH0 * C].reshape(NS, CH0, C)
        p1 = x[NS * CH0 * C:].reshape(NS, CH1, C)
        p0 = jnp.pad(p0, ((0, 0), (0, CHm - CH0), (0, 0)))
        p1 = jnp.pad(p1, ((0, 0), (0, CHm - CH1), (0, 0)))
        return jnp.concatenate([p0, p1], axis=0)

    ep = jnp.stack([_slab(colf), _slab(rowf), _slab(wbits)], axis=2)
    ep = jnp.pad(ep, ((0, 0), (0, 0), (0, 5), (0, 0))).reshape(
        T * CHm * 8, C)

    CHd = max(4, 4 * (-(-E // (T * C * 4))))
    padd = T * CHd * C - E
    col = jnp.pad(edge_index[1], (0, padd)).reshape(T, CHd, C)
    w = jnp.pad(edge_weight, (0, padd)).reshape(T, CHd, C)

    rpt = -(-N // NS)
    Np = NS * (-(-rpt // 128) * 128)
    degp = _make_deg(Np, NC, NS, CHd, C)(col, w)
    dg, dinv = _tc_prep(degp[0, :N, None], degp[1, :N, None])

    BN = 1000 if N % 1000 == 0 else 8
    spmm = _make_spmm(Np, d, NC, NS, CHm, C, CH0, CH1)

    branches = []
    g = _tc_scale(X, dg, BN)
    for _ in range(3):
        p = spmm(g, ep)
        g, br = _tc_combine_gcn(p[0, :N], p[1, :N], g, dg, BN)
        branches.append(br)

    m = jnp.asarray(moment, jnp.float32).reshape(1, 1)
    fp = X
    dix = _tc_scale(X, dinv, BN)
    for t in range(4):
        p = spmm(dix, ep)
        fp, dix, br = _tc_combine_sct(p[0, :N], p[1, :N], fp, dinv, m, BN)
        if t > 0:
            branches.append(br)

    b1r = b1.reshape(1, d)
    b2r = b2.reshape(1, d)
    return _tc_attend(X, branches, a, W1, b1r, W2, b2r, BN)


# speedup vs baseline: 1.4547x; 1.4547x over previous
import functools

import jax
import jax.numpy as jnp
from jax import lax
from jax.experimental import pallas as pl
from jax.experimental.pallas import tpu as pltpu
from jax.experimental.pallas import tpu_sc as plsc


def _leaky(x, slope=0.01):
    return jnp.where(x > 0, x, slope * x)


def _sc_geometry():
    try:
        info = plsc.get_sparse_core_info()
        return info.num_cores, info.num_subcores, info.num_lanes
    except Exception:
        return 2, 16, 16


def _make_spmm(Np, d, NC, NS, CH, C, CH0, CH1):
    mesh = plsc.VectorSubcoreMesh(core_axis_name="c", subcore_axis_name="s")
    rpt = Np // NS
    BC = min(128, C)
    nb = rpt // BC
    nk = d // 16

    @functools.partial(
        pl.kernel,
        out_type=jax.ShapeDtypeStruct((NC, Np, d), jnp.float32),
        mesh=mesh,
        scratch_types=[
            pltpu.VMEM((CH * C // 128, 128), jnp.int32),
            pltpu.VMEM((CH * C // 128, 128), jnp.float32),
            pltpu.VMEM((4, C), jnp.int32),
            pltpu.VMEM((4, C), jnp.int32),
            pltpu.VMEM((C, d), jnp.float32),
            pltpu.VMEM((C, d), jnp.float32),
            pltpu.VMEM((C, d), jnp.float32),
            pltpu.VMEM((C, d), jnp.float32),
            pltpu.VMEM_SHARED((Np, d), jnp.float32),
            pltpu.SemaphoreType.DMA,
            pltpu.SemaphoreType.DMA,
            pltpu.SemaphoreType.DMA,
            pltpu.SemaphoreType.DMA,
            pltpu.SemaphoreType.DMA,
            pltpu.SemaphoreType.DMA,
            pltpu.SemaphoreType.DMA,
            pltpu.SemaphoreType.DMA,
        ],
    )
    def spmm(xs, rcp, wp, out, rc_v, w_v, colix, rowix,
             g0, g1, g2, g3, acc, gm0, gm1, gm2, gm3, sm0, sm1, sm2, sm3):
        cid = lax.axis_index("c")
        sid = lax.axis_index("s")
        wid = cid * NS + sid
        pltpu.async_copy(rcp.at[wid], rc_v, gm0)
        pltpu.async_copy(wp.at[wid], w_v, gm1)

        gbufs = (g0, g1, g2, g3)
        gsems = (gm0, gm1, gm2, gm3)
        ssems = (sm0, sm1, sm2, sm3)
        zeros16 = jnp.zeros((16,), jnp.float32)

        def zrow(i, _):
            for k in range(nk):
                g0[i, pl.ds(k * 16, 16)] = zeros16
            return 0

        lax.fori_loop(0, BC, zrow, 0)
        r0 = sid * rpt

        def zacc(j, _):
            pltpu.sync_copy(g0.at[pl.ds(0, BC)],
                            acc.at[pl.ds(r0 + j * BC, BC)])
            return 0

        lax.fori_loop(0, nb, zacc, 0)
        pltpu.make_async_copy(rcp.at[wid], rc_v, gm0).wait()
        pltpu.make_async_copy(wp.at[wid], w_v, gm1).wait()
        plsc.subcore_barrier()

        def unpack_col(ch, b):
            q = ch // 4
            r = (ch % 4) * C

            def ug(g, _):
                v = rc_v[q, pl.ds(r + g * 16, 16)]
                colix[b, pl.ds(g * 16, 16)] = lax.bitwise_and(v, 16383)
                return 0

            lax.fori_loop(0, C // 16, ug, 0)

        def unpack_row(ch, b):
            q = ch // 4
            r = (ch % 4) * C

            def ug(g, _):
                v = rc_v[q, pl.ds(r + g * 16, 16)]
                rowix[b, pl.ds(g * 16, 16)] = lax.shift_right_logical(v, 14)
                return 0

            lax.fori_loop(0, C // 16, ug, 0)

        def start_gather(ch, b):
            pltpu.async_copy(xs.at[colix.at[b]], gbufs[b], gsems[b])

        def wait_gather(b):
            pltpu.make_async_copy(xs.at[colix.at[b]], gbufs[b],
                                  gsems[b]).wait()

        def start_scatter(b):
            pltpu.async_copy(gbufs[b], acc.at[rowix.at[b]], ssems[b],
                             add=True)

        def wait_scatter(b):
            pltpu.make_async_copy(gbufs[b], acc.at[rowix.at[b]],
                                  ssems[b]).wait()

        def scale(ch, b):
            gb = gbufs[b]
            q = ch // 4
            r = (ch % 4) * C

            def egroup(g, _):
                wv = w_v[q, pl.ds(r + g * 16, 16)]
                for j in range(16):
                    wb = jnp.broadcast_to(wv[j], (16,))
                    e = g * 16 + j
                    for k in range(nk):
                        sl = pl.ds(k * 16, 16)
                        gb[e, sl] = gb[e, sl] * wb
                return 0

            lax.fori_loop(0, C // 16, egroup, 0)

        nch = jnp.where(cid == 0, CH0, CH1)

        for b in range(3):
            unpack_col(b, b)
            start_gather(b, b)
        for ch in range(4):
            b = ch
            wait_gather(b)
            unpack_row(ch, b)
            scale(ch, b)
            start_scatter(b)
            nb2 = (b + 3) % 4
            if ch > 0:
                wait_scatter(nb2)
            unpack_col(ch + 3, nb2)
            start_gather(ch + 3, nb2)

        def quad(g, _):
            for boff in range(4):
                ch = 4 * g + boff
                b = boff
                wait_gather(b)
                unpack_row(ch, b)
                scale(ch, b)
                start_scatter(b)
                nb2 = (b + 3) % 4

                @pl.when(ch + 3 < nch)
                def _():
                    wait_scatter(nb2)
                    unpack_col(ch + 3, nb2)
                    start_gather(ch + 3, nb2)
            return 0

        lax.fori_loop(1, nch // 4, quad, 0)
        for b in range(4):
            wait_scatter(b)
        plsc.subcore_barrier()

        def wout(j, _):
            pltpu.sync_copy(acc.at[pl.ds(r0 + j * BC, BC)],
                            g0.at[pl.ds(0, BC)])
            pltpu.sync_copy(g0.at[pl.ds(0, BC)],
                            out.at[cid, pl.ds(r0 + j * BC, BC)])
            return 0

        lax.fori_loop(0, nb, wout, 0)

    return spmm


def _make_deg(Np, NC, NS, CH, C):
    mesh = plsc.VectorSubcoreMesh(core_axis_name="c", subcore_axis_name="s")
    rpt = Np // NS

    @functools.partial(
        pl.kernel,
        out_type=jax.ShapeDtypeStruct((NC, Np), jnp.float32),
        mesh=mesh,
        scratch_types=[
            pltpu.VMEM((CH, C), jnp.int32),
            pltpu.VMEM((CH, C), jnp.float32),
            pltpu.VMEM((rpt,), jnp.float32),
            pltpu.VMEM_SHARED((Np,), jnp.float32),
            pltpu.SemaphoreType.DMA,
            pltpu.SemaphoreType.DMA,
            pltpu.SemaphoreType.DMA,
            pltpu.SemaphoreType.DMA,
        ],
    )
    def deg(colp, wp, out, col_v, w_v, dbuf, acc, dm0, dm1, dm2, dm3):
        cid = lax.axis_index("c")
        sid = lax.axis_index("s")
        wid = cid * NS + sid
        pltpu.sync_copy(colp.at[wid], col_v)
        pltpu.sync_copy(wp.at[wid], w_v)

        zeros16 = jnp.zeros((16,), jnp.float32)

        def zrow(i, _):
            dbuf[pl.ds(i * 16, 16)] = zeros16
            return 0

        lax.fori_loop(0, rpt // 16, zrow, 0)
        r0 = sid * rpt
        pltpu.sync_copy(dbuf, acc.at[pl.ds(r0, rpt)])
        plsc.subcore_barrier()

        dsems = (dm0, dm1, dm2, dm3)

        def dstart(ch, b):
            pltpu.async_copy(w_v.at[ch], acc.at[col_v.at[ch]], dsems[b],
                             add=True)

        def dwait(b):
            pltpu.make_async_copy(w_v.at[0], acc.at[col_v.at[0]],
                                  dsems[b]).wait()

        for ch in range(4):
            dstart(ch, ch)

        def quad(g, _):
            for b in range(4):
                ch = 4 * g + b
                dwait(b)
                dstart(ch, b)
            return 0

        lax.fori_loop(1, CH // 4, quad, 0)
        for b in range(4):
            dwait(b)
        plsc.subcore_barrier()
        pltpu.sync_copy(acc.at[pl.ds(r0, rpt)], dbuf)
        pltpu.sync_copy(dbuf, out.at[cid, pl.ds(r0, rpt)])

    return deg


def _prep_body(d0_ref, d1_ref, dg_ref, dinv_ref):
    deg = d0_ref[...] + d1_ref[...]
    dg_ref[...] = lax.rsqrt(deg + 1.0)
    dinv_ref[...] = jnp.where(deg > 0, 1.0 / deg, 0.0)


def _tc_prep(d0, d1):
    N = d0.shape[0]
    return pl.pallas_call(
        _prep_body,
        out_shape=(jax.ShapeDtypeStruct((N, 1), jnp.float32),
                   jax.ShapeDtypeStruct((N, 1), jnp.float32)),
    )(d0, d1)


def _scale_body(f_ref, s_ref, o_ref):
    o_ref[...] = f_ref[...] * s_ref[...]


def _tc_scale(f, s, BN):
    N, d = f.shape
    return pl.pallas_call(
        _scale_body,
        grid=(N // BN,),
        in_specs=[pl.BlockSpec((BN, d), lambda i: (i, 0)),
                  pl.BlockSpec((BN, 1), lambda i: (i, 0))],
        out_specs=pl.BlockSpec((BN, d), lambda i: (i, 0)),
        out_shape=jax.ShapeDtypeStruct((N, d), jnp.float32),
    )(f, s)


def _gcn_body(p0_ref, p1_ref, g_ref, dg_ref, gn_ref, br_ref):
    fn = dg_ref[...] * (p0_ref[...] + p1_ref[...] + g_ref[...])
    gn_ref[...] = dg_ref[...] * fn
    br_ref[...] = _leaky(fn)


def _tc_combine_gcn(p0, p1, g, dg, BN):
    N, d = g.shape
    return pl.pallas_call(
        _gcn_body,
        grid=(N // BN,),
        in_specs=[pl.BlockSpec((BN, d), lambda i: (i, 0)),
                  pl.BlockSpec((BN, d), lambda i: (i, 0)),
                  pl.BlockSpec((BN, d), lambda i: (i, 0)),
                  pl.BlockSpec((BN, 1), lambda i: (i, 0))],
        out_specs=(pl.BlockSpec((BN, d), lambda i: (i, 0)),
                   pl.BlockSpec((BN, d), lambda i: (i, 0))),
        out_shape=(jax.ShapeDtypeStruct((N, d), jnp.float32),
                   jax.ShapeDtypeStruct((N, d), jnp.float32)),
    )(p0, p1, g, dg)


def _sct_body(p0_ref, p1_ref, fp_ref, dinv_ref, m_ref, fn_ref, dx_ref, br_ref):
    fn = 0.5 * fp_ref[...] + 0.5 * (p0_ref[...] + p1_ref[...])
    fn_ref[...] = fn
    dx_ref[...] = dinv_ref[...] * fn
    ad = jnp.abs(fp_ref[...] - fn)
    m = m_ref[...]
    br_ref[...] = jnp.where(
        ad > 0, jnp.exp(m * jnp.log(jnp.maximum(ad, 1e-38))), 0.0)


def _tc_combine_sct(p0, p1, fp, dinv, m, BN):
    N, d = fp.shape
    return pl.pallas_call(
        _sct_body,
        grid=(N // BN,),
        in_specs=[pl.BlockSpec((BN, d), lambda i: (i, 0)),
                  pl.BlockSpec((BN, d), lambda i: (i, 0)),
                  pl.BlockSpec((BN, d), lambda i: (i, 0)),
                  pl.BlockSpec((BN, 1), lambda i: (i, 0)),
                  pl.BlockSpec((1, 1), lambda i: (0, 0))],
        out_specs=(pl.BlockSpec((BN, d), lambda i: (i, 0)),
                   pl.BlockSpec((BN, d), lambda i: (i, 0)),
                   pl.BlockSpec((BN, d), lambda i: (i, 0))),
        out_shape=(jax.ShapeDtypeStruct((N, d), jnp.float32),
                   jax.ShapeDtypeStruct((N, d), jnp.float32),
                   jax.ShapeDtypeStruct((N, d), jnp.float32)),
    )(p0, p1, fp, dinv, m)


def _attend_body(x_ref, h0, h1, h2, h3, h4, h5, a_ref, w1_ref, b1_ref,
                 w2_ref, b2_ref, o_ref):
    d = x_ref.shape[1]
    a1 = a_ref[pl.ds(0, d), :]
    a2 = a_ref[pl.ds(d, d), :]
    c = jnp.dot(jnp.maximum(x_ref[...], 0.0), a1,
                preferred_element_type=jnp.float32)
    hs = [h0[...], h1[...], h2[...], h3[...], h4[...], h5[...]]
    es = [c + jnp.dot(jnp.maximum(h, 0.0), a2,
                      preferred_element_type=jnp.float32) for h in hs]
    e = jnp.concatenate(es, axis=1)
    e = e - jnp.max(e, axis=1, keepdims=True)
    ex = jnp.exp(e)
    att = ex / jnp.sum(ex, axis=1, keepdims=True)
    hp = att[:, 0:1] * hs[0]
    for k in range(1, 6):
        hp = hp + att[:, k:k + 1] * hs[k]
    hp = hp * (1.0 / 6.0)
    t = _leaky(lax.dot_general(hp, w1_ref[...], (((1,), (1,)), ((), ())),
                               preferred_element_type=jnp.float32)
               + b1_ref[...])
    o_ref[...] = _leaky(
        lax.dot_general(t, w2_ref[...], (((1,), (1,)), ((), ())),
                        preferred_element_type=jnp.float32) + b2_ref[...])


def _tc_attend(x, hs, a, w1, b1, w2, b2, BN):
    N, d = x.shape
    blk = pl.BlockSpec((BN, d), lambda i: (i, 0))
    return pl.pallas_call(
        _attend_body,
        grid=(N // BN,),
        in_specs=[blk, blk, blk, blk, blk, blk, blk,
                  pl.BlockSpec((2 * d, 1), lambda i: (0, 0)),
                  pl.BlockSpec((d, d), lambda i: (0, 0)),
                  pl.BlockSpec((1, d), lambda i: (0, 0)),
                  pl.BlockSpec((d, d), lambda i: (0, 0)),
                  pl.BlockSpec((1, d), lambda i: (0, 0))],
        out_specs=blk,
        out_shape=jax.ShapeDtypeStruct((N, d), jnp.float32),
    )(x, *hs, a, w1, b1, w2, b2)


def kernel(X, edge_index, edge_weight, W1, b1, W2, b2, a, moment):
    N, d = X.shape
    E = edge_weight.shape[0]
    NC, NS, _ = _sc_geometry()
    T = NC * NS
    C = 32
    tot = max(8, 4 * (-(-E // (NS * C * 4))))
    FR0 = 0.63
    CH0 = max(4, 4 * round(tot * FR0 / 4))
    CH1 = tot - CH0
    CHm = max(CH0, CH1)
    pad = NS * tot * C - E
    rowf = jnp.pad(edge_index[0], (0, pad))
    colf = jnp.pad(edge_index[1], (0, pad))
    wf = jnp.pad(edge_weight, (0, pad))
    rcf = jnp.bitwise_or(jnp.left_shift(rowf, 14), colf)

    def _slab(x):
        p0 = x[:NS * CH0 * C].reshape(NS, CH0, C)
        p1 = x[NS * CH0 * C:].reshape(NS, CH1, C)
        p0 = jnp.pad(p0, ((0, 0), (0, CHm - CH0), (0, 0)))
        p1 = jnp.pad(p1, ((0, 0), (0, CHm - CH1), (0, 0)))
        return jnp.concatenate([p0, p1], axis=0)

    rc = _slab(rcf).reshape(T, CHm * C // 128, 128)
    w_s = _slab(wf).reshape(T, CHm * C // 128, 128)

    CHd = max(4, 4 * (-(-E // (T * C * 4))))
    padd = T * CHd * C - E
    col = jnp.pad(edge_index[1], (0, padd)).reshape(T, CHd, C)
    w = jnp.pad(edge_weight, (0, padd)).reshape(T, CHd, C)

    rpt = -(-N // NS)
    Np = NS * (-(-rpt // 128) * 128)
    degp = _make_deg(Np, NC, NS, CHd, C)(col, w)
    dg, dinv = _tc_prep(degp[0, :N, None], degp[1, :N, None])

    BN = 1000 if N % 1000 == 0 else 8
    spmm = _make_spmm(Np, d, NC, NS, CHm, C, CH0, CH1)

    branches = []
    g = _tc_scale(X, dg, BN)
    for _ in range(3):
        p = spmm(g, rc, w_s)
        g, br = _tc_combine_gcn(p[0, :N], p[1, :N], g, dg, BN)
        branches.append(br)

    m = jnp.asarray(moment, jnp.float32).reshape(1, 1)
    fp = X
    dix = _tc_scale(X, dinv, BN)
    for t in range(4):
        p = spmm(dix, rc, w_s)
        fp, dix, br = _tc_combine_sct(p[0, :N], p[1, :N], fp, dinv, m, BN)
        if t > 0:
            branches.append(br)

    b1r = b1.reshape(1, d)
    b2r = b2.reshape(1, d)
    return _tc_attend(X, branches, a, W1, b1r, W2, b2r, BN)


# speedup vs baseline: 1.4554x; 1.0005x over previous
import functools

import jax
import jax.numpy as jnp
from jax import lax
from jax.experimental import pallas as pl
from jax.experimental.pallas import tpu as pltpu
from jax.experimental.pallas import tpu_sc as plsc


def _leaky(x, slope=0.01):
    return jnp.where(x > 0, x, slope * x)


def _sc_geometry():
    try:
        info = plsc.get_sparse_core_info()
        return info.num_cores, info.num_subcores, info.num_lanes
    except Exception:
        return 2, 16, 16


def _make_spmm(Np, d, NC, NS, CH, C, CH0, CH1):
    mesh = plsc.VectorSubcoreMesh(core_axis_name="c", subcore_axis_name="s")
    rpt = Np // NS
    BC = min(128, C)
    nb = rpt // BC
    nk = d // 16

    @functools.partial(
        pl.kernel,
        out_type=jax.ShapeDtypeStruct((NC, Np, d), jnp.float32),
        mesh=mesh,
        scratch_types=[
            pltpu.VMEM((CH * C // 128, 128), jnp.int32),
            pltpu.VMEM((CH * C // 128, 128), jnp.float32),
            pltpu.VMEM((4, C), jnp.int32),
            pltpu.VMEM((4, C), jnp.int32),
            pltpu.VMEM((C, d), jnp.float32),
            pltpu.VMEM((C, d), jnp.float32),
            pltpu.VMEM((C, d), jnp.float32),
            pltpu.VMEM((C, d), jnp.float32),
            pltpu.VMEM_SHARED((Np, d), jnp.float32),
            pltpu.SemaphoreType.DMA,
            pltpu.SemaphoreType.DMA,
            pltpu.SemaphoreType.DMA,
            pltpu.SemaphoreType.DMA,
            pltpu.SemaphoreType.DMA,
            pltpu.SemaphoreType.DMA,
            pltpu.SemaphoreType.DMA,
            pltpu.SemaphoreType.DMA,
        ],
    )
    def spmm(xs, rcp, wp, out, rc_v, w_v, colix, rowix,
             g0, g1, g2, g3, acc, gm0, gm1, gm2, gm3, sm0, sm1, sm2, sm3):
        cid = lax.axis_index("c")
        sid = lax.axis_index("s")
        wid = cid * NS + sid
        pltpu.async_copy(rcp.at[wid], rc_v, gm0)
        pltpu.async_copy(wp.at[wid], w_v, gm1)

        gbufs = (g0, g1, g2, g3)
        gsems = (gm0, gm1, gm2, gm3)
        ssems = (sm0, sm1, sm2, sm3)
        zeros16 = jnp.zeros((16,), jnp.float32)

        def zrow(i, _):
            for k in range(nk):
                g0[i, pl.ds(k * 16, 16)] = zeros16
            return 0

        lax.fori_loop(0, BC, zrow, 0)
        r0 = sid * rpt

        def zacc(j, _):
            pltpu.sync_copy(g0.at[pl.ds(0, BC)],
                            acc.at[pl.ds(r0 + j * BC, BC)])
            return 0

        lax.fori_loop(0, nb, zacc, 0)
        pltpu.make_async_copy(rcp.at[wid], rc_v, gm0).wait()
        pltpu.make_async_copy(wp.at[wid], w_v, gm1).wait()
        plsc.subcore_barrier()

        def unpack_col(ch, b):
            q = ch // 4
            r = (ch % 4) * C

            def ug(g, _):
                v = rc_v[q, pl.ds(r + g * 16, 16)]
                colix[b, pl.ds(g * 16, 16)] = lax.bitwise_and(v, 16383)
                return 0

            lax.fori_loop(0, C // 16, ug, 0)

        def unpack_row(ch, b):
            q = ch // 4
            r = (ch % 4) * C

            def ug(g, _):
                v = rc_v[q, pl.ds(r + g * 16, 16)]
                rowix[b, pl.ds(g * 16, 16)] = lax.shift_right_logical(v, 14)
                return 0

            lax.fori_loop(0, C // 16, ug, 0)

        def start_gather(ch, b):
            pltpu.async_copy(xs.at[colix.at[b, pl.ds(0, 16)]],
                             gbufs[b].at[pl.ds(0, 16)], gsems[b])
            pltpu.async_copy(xs.at[colix.at[b, pl.ds(16, 16)]],
                             gbufs[b].at[pl.ds(16, 16)], gsems[b])

        def wait_gather(b):
            pltpu.make_async_copy(xs.at[colix.at[b, pl.ds(0, 16)]],
                                  gbufs[b].at[pl.ds(0, 16)],
                                  gsems[b]).wait()
            pltpu.make_async_copy(xs.at[colix.at[b, pl.ds(16, 16)]],
                                  gbufs[b].at[pl.ds(16, 16)],
                                  gsems[b]).wait()

        def start_scatter(b):
            pltpu.async_copy(gbufs[b], acc.at[rowix.at[b]], ssems[b],
                             add=True)

        def wait_scatter(b):
            pltpu.make_async_copy(gbufs[b], acc.at[rowix.at[b]],
                                  ssems[b]).wait()

        def scale(ch, b):
            gb = gbufs[b]
            q = ch // 4
            r = (ch % 4) * C

            def egroup(g, _):
                wv = w_v[q, pl.ds(r + g * 16, 16)]
                for j in range(16):
                    wb = jnp.broadcast_to(wv[j], (16,))
                    e = g * 16 + j
                    for k in range(nk):
                        sl = pl.ds(k * 16, 16)
                        gb[e, sl] = gb[e, sl] * wb
                return 0

            lax.fori_loop(0, C // 16, egroup, 0)

        nch = jnp.where(cid == 0, CH0, CH1)

        for b in range(3):
            unpack_col(b, b)
            start_gather(b, b)
        for ch in range(4):
            b = ch
            wait_gather(b)
            unpack_row(ch, b)
            scale(ch, b)
            start_scatter(b)
            nb2 = (b + 3) % 4
            if ch > 0:
                wait_scatter(nb2)
            unpack_col(ch + 3, nb2)
            start_gather(ch + 3, nb2)

        def quad(g, _):
            for boff in range(4):
                ch = 4 * g + boff
                b = boff
                wait_gather(b)
                unpack_row(ch, b)
                scale(ch, b)
                start_scatter(b)
                nb2 = (b + 3) % 4

                @pl.when(ch + 3 < nch)
                def _():
                    wait_scatter(nb2)
                    unpack_col(ch + 3, nb2)
                    start_gather(ch + 3, nb2)
            return 0

        lax.fori_loop(1, nch // 4, quad, 0)
        for b in range(4):
            wait_scatter(b)
        plsc.subcore_barrier()

        def wout(j, _):
            pltpu.sync_copy(acc.at[pl.ds(r0 + j * BC, BC)],
                            g0.at[pl.ds(0, BC)])
            pltpu.sync_copy(g0.at[pl.ds(0, BC)],
                            out.at[cid, pl.ds(r0 + j * BC, BC)])
            return 0

        lax.fori_loop(0, nb, wout, 0)

    return spmm


def _make_deg(Np, NC, NS, CH, C):
    mesh = plsc.VectorSubcoreMesh(core_axis_name="c", subcore_axis_name="s")
    rpt = Np // NS

    @functools.partial(
        pl.kernel,
        out_type=jax.ShapeDtypeStruct((NC, Np), jnp.float32),
        mesh=mesh,
        scratch_types=[
            pltpu.VMEM((CH, C), jnp.int32),
            pltpu.VMEM((CH, C), jnp.float32),
            pltpu.VMEM((rpt,), jnp.float32),
            pltpu.VMEM_SHARED((Np,), jnp.float32),
            pltpu.SemaphoreType.DMA,
            pltpu.SemaphoreType.DMA,
            pltpu.SemaphoreType.DMA,
            pltpu.SemaphoreType.DMA,
        ],
    )
    def deg(colp, wp, out, col_v, w_v, dbuf, acc, dm0, dm1, dm2, dm3):
        cid = lax.axis_index("c")
        sid = lax.axis_index("s")
        wid = cid * NS + sid
        pltpu.sync_copy(colp.at[wid], col_v)
        pltpu.sync_copy(wp.at[wid], w_v)

        zeros16 = jnp.zeros((16,), jnp.float32)

        def zrow(i, _):
            dbuf[pl.ds(i * 16, 16)] = zeros16
            return 0

        lax.fori_loop(0, rpt // 16, zrow, 0)
        r0 = sid * rpt
        pltpu.sync_copy(dbuf, acc.at[pl.ds(r0, rpt)])
        plsc.subcore_barrier()

        dsems = (dm0, dm1, dm2, dm3)

        def dstart(ch, b):
            pltpu.async_copy(w_v.at[ch], acc.at[col_v.at[ch]], dsems[b],
                             add=True)

        def dwait(b):
            pltpu.make_async_copy(w_v.at[0], acc.at[col_v.at[0]],
                                  dsems[b]).wait()

        for ch in range(4):
            dstart(ch, ch)

        def quad(g, _):
            for b in range(4):
                ch = 4 * g + b
                dwait(b)
                dstart(ch, b)
            return 0

        lax.fori_loop(1, CH // 4, quad, 0)
        for b in range(4):
            dwait(b)
        plsc.subcore_barrier()
        pltpu.sync_copy(acc.at[pl.ds(r0, rpt)], dbuf)
        pltpu.sync_copy(dbuf, out.at[cid, pl.ds(r0, rpt)])

    return deg


def _prep_body(d0_ref, d1_ref, dg_ref, dinv_ref):
    deg = d0_ref[...] + d1_ref[...]
    dg_ref[...] = lax.rsqrt(deg + 1.0)
    dinv_ref[...] = jnp.where(deg > 0, 1.0 / deg, 0.0)


def _tc_prep(d0, d1):
    N = d0.shape[0]
    return pl.pallas_call(
        _prep_body,
        out_shape=(jax.ShapeDtypeStruct((N, 1), jnp.float32),
                   jax.ShapeDtypeStruct((N, 1), jnp.float32)),
    )(d0, d1)


def _scale_body(f_ref, s_ref, o_ref):
    o_ref[...] = f_ref[...] * s_ref[...]


def _tc_scale(f, s, BN):
    N, d = f.shape
    return pl.pallas_call(
        _scale_body,
        grid=(N // BN,),
        in_specs=[pl.BlockSpec((BN, d), lambda i: (i, 0)),
                  pl.BlockSpec((BN, 1), lambda i: (i, 0))],
        out_specs=pl.BlockSpec((BN, d), lambda i: (i, 0)),
        out_shape=jax.ShapeDtypeStruct((N, d), jnp.float32),
    )(f, s)


def _gcn_body(p0_ref, p1_ref, g_ref, dg_ref, gn_ref, br_ref):
    fn = dg_ref[...] * (p0_ref[...] + p1_ref[...] + g_ref[...])
    gn_ref[...] = dg_ref[...] * fn
    br_ref[...] = _leaky(fn)


def _tc_combine_gcn(p0, p1, g, dg, BN):
    N, d = g.shape
    return pl.pallas_call(
        _gcn_body,
        grid=(N // BN,),
        in_specs=[pl.BlockSpec((BN, d), lambda i: (i, 0)),
                  pl.BlockSpec((BN, d), lambda i: (i, 0)),
                  pl.BlockSpec((BN, d), lambda i: (i, 0)),
                  pl.BlockSpec((BN, 1), lambda i: (i, 0))],
        out_specs=(pl.BlockSpec((BN, d), lambda i: (i, 0)),
                   pl.BlockSpec((BN, d), lambda i: (i, 0))),
        out_shape=(jax.ShapeDtypeStruct((N, d), jnp.float32),
                   jax.ShapeDtypeStruct((N, d), jnp.float32)),
    )(p0, p1, g, dg)


def _sct_body(p0_ref, p1_ref, fp_ref, dinv_ref, m_ref, fn_ref, dx_ref, br_ref):
    fn = 0.5 * fp_ref[...] + 0.5 * (p0_ref[...] + p1_ref[...])
    fn_ref[...] = fn
    dx_ref[...] = dinv_ref[...] * fn
    ad = jnp.abs(fp_ref[...] - fn)
    m = m_ref[...]
    br_ref[...] = jnp.where(
        ad > 0, jnp.exp(m * jnp.log(jnp.maximum(ad, 1e-38))), 0.0)


def _tc_combine_sct(p0, p1, fp, dinv, m, BN):
    N, d = fp.shape
    return pl.pallas_call(
        _sct_body,
        grid=(N // BN,),
        in_specs=[pl.BlockSpec((BN, d), lambda i: (i, 0)),
                  pl.BlockSpec((BN, d), lambda i: (i, 0)),
                  pl.BlockSpec((BN, d), lambda i: (i, 0)),
                  pl.BlockSpec((BN, 1), lambda i: (i, 0)),
                  pl.BlockSpec((1, 1), lambda i: (0, 0))],
        out_specs=(pl.BlockSpec((BN, d), lambda i: (i, 0)),
                   pl.BlockSpec((BN, d), lambda i: (i, 0)),
                   pl.BlockSpec((BN, d), lambda i: (i, 0))),
        out_shape=(jax.ShapeDtypeStruct((N, d), jnp.float32),
                   jax.ShapeDtypeStruct((N, d), jnp.float32),
                   jax.ShapeDtypeStruct((N, d), jnp.float32)),
    )(p0, p1, fp, dinv, m)


def _attend_body(x_ref, h0, h1, h2, h3, h4, h5, a_ref, w1_ref, b1_ref,
                 w2_ref, b2_ref, o_ref):
    d = x_ref.shape[1]
    a1 = a_ref[pl.ds(0, d), :]
    a2 = a_ref[pl.ds(d, d), :]
    c = jnp.dot(jnp.maximum(x_ref[...], 0.0), a1,
                preferred_element_type=jnp.float32)
    hs = [h0[...], h1[...], h2[...], h3[...], h4[...], h5[...]]
    es = [c + jnp.dot(jnp.maximum(h, 0.0), a2,
                      preferred_element_type=jnp.float32) for h in hs]
    e = jnp.concatenate(es, axis=1)
    e = e - jnp.max(e, axis=1, keepdims=True)
    ex = jnp.exp(e)
    att = ex / jnp.sum(ex, axis=1, keepdims=True)
    hp = att[:, 0:1] * hs[0]
    for k in range(1, 6):
        hp = hp + att[:, k:k + 1] * hs[k]
    hp = hp * (1.0 / 6.0)
    t = _leaky(lax.dot_general(hp, w1_ref[...], (((1,), (1,)), ((), ())),
                               preferred_element_type=jnp.float32)
               + b1_ref[...])
    o_ref[...] = _leaky(
        lax.dot_general(t, w2_ref[...], (((1,), (1,)), ((), ())),
                        preferred_element_type=jnp.float32) + b2_ref[...])


def _tc_attend(x, hs, a, w1, b1, w2, b2, BN):
    N, d = x.shape
    blk = pl.BlockSpec((BN, d), lambda i: (i, 0))
    return pl.pallas_call(
        _attend_body,
        grid=(N // BN,),
        in_specs=[blk, blk, blk, blk, blk, blk, blk,
                  pl.BlockSpec((2 * d, 1), lambda i: (0, 0)),
                  pl.BlockSpec((d, d), lambda i: (0, 0)),
                  pl.BlockSpec((1, d), lambda i: (0, 0)),
                  pl.BlockSpec((d, d), lambda i: (0, 0)),
                  pl.BlockSpec((1, d), lambda i: (0, 0))],
        out_specs=blk,
        out_shape=jax.ShapeDtypeStruct((N, d), jnp.float32),
    )(x, *hs, a, w1, b1, w2, b2)


def kernel(X, edge_index, edge_weight, W1, b1, W2, b2, a, moment):
    N, d = X.shape
    E = edge_weight.shape[0]
    NC, NS, _ = _sc_geometry()
    T = NC * NS
    C = 32
    tot = max(8, 4 * (-(-E // (NS * C * 4))))
    FR0 = 0.63
    CH0 = max(4, 4 * round(tot * FR0 / 4))
    CH1 = tot - CH0
    CHm = max(CH0, CH1)
    pad = NS * tot * C - E
    rowf = jnp.pad(edge_index[0], (0, pad))
    colf = jnp.pad(edge_index[1], (0, pad))
    wf = jnp.pad(edge_weight, (0, pad))
    rcf = jnp.bitwise_or(jnp.left_shift(rowf, 14), colf)

    def _slab(x):
        p0 = x[:NS * CH0 * C].reshape(NS, CH0, C)
        p1 = x[NS * CH0 * C:].reshape(NS, CH1, C)
        p0 = jnp.pad(p0, ((0, 0), (0, CHm - CH0), (0, 0)))
        p1 = jnp.pad(p1, ((0, 0), (0, CHm - CH1), (0, 0)))
        return jnp.concatenate([p0, p1], axis=0)

    rc = _slab(rcf).reshape(T, CHm * C // 128, 128)
    w_s = _slab(wf).reshape(T, CHm * C // 128, 128)

    CHd = max(4, 4 * (-(-E // (T * C * 4))))
    padd = T * CHd * C - E
    col = jnp.pad(edge_index[1], (0, padd)).reshape(T, CHd, C)
    w = jnp.pad(edge_weight, (0, padd)).reshape(T, CHd, C)

    rpt = -(-N // NS)
    Np = NS * (-(-rpt // 128) * 128)
    degp = _make_deg(Np, NC, NS, CHd, C)(col, w)
    dg, dinv = _tc_prep(degp[0, :N, None], degp[1, :N, None])

    BN = 1000 if N % 1000 == 0 else 8
    spmm = _make_spmm(Np, d, NC, NS, CHm, C, CH0, CH1)

    branches = []
    g = _tc_scale(X, dg, BN)
    for _ in range(3):
        p = spmm(g, rc, w_s)
        g, br = _tc_combine_gcn(p[0, :N], p[1, :N], g, dg, BN)
        branches.append(br)

    m = jnp.asarray(moment, jnp.float32).reshape(1, 1)
    fp = X
    dix = _tc_scale(X, dinv, BN)
    for t in range(4):
        p = spmm(dix, rc, w_s)
        fp, dix, br = _tc_combine_sct(p[0, :N], p[1, :N], fp, dinv, m, BN)
        if t > 0:
            branches.append(br)

    b1r = b1.reshape(1, d)
    b2r = b2.reshape(1, d)
    return _tc_attend(X, branches, a, W1, b1r, W2, b2r, BN)


# speedup vs baseline: 1.4686x; 1.0091x over previous
import functools

import jax
import jax.numpy as jnp
from jax import lax
from jax.experimental import pallas as pl
from jax.experimental.pallas import tpu as pltpu
from jax.experimental.pallas import tpu_sc as plsc


def _leaky(x, slope=0.01):
    return jnp.where(x > 0, x, slope * x)


def _sc_geometry():
    try:
        info = plsc.get_sparse_core_info()
        return info.num_cores, info.num_subcores, info.num_lanes
    except Exception:
        return 2, 16, 16


def _make_spmm(Np, d, NC, NS, CH, C, CH0, CH1):
    mesh = plsc.VectorSubcoreMesh(core_axis_name="c", subcore_axis_name="s")
    rpt = Np // NS
    BC = min(128, C)
    nb = rpt // BC
    nk = d // 16

    @functools.partial(
        pl.kernel,
        out_type=jax.ShapeDtypeStruct((NC, Np, d), jnp.float32),
        mesh=mesh,
        scratch_types=[
            pltpu.VMEM((CH * C // 128, 128), jnp.int32),
            pltpu.VMEM((CH * C // 128, 128), jnp.float32),
            pltpu.VMEM((4, C), jnp.int32),
            pltpu.VMEM((4, C), jnp.int32),
            pltpu.VMEM((C, d), jnp.float32),
            pltpu.VMEM((C, d), jnp.float32),
            pltpu.VMEM((C, d), jnp.float32),
            pltpu.VMEM((C, d), jnp.float32),
            pltpu.VMEM_SHARED((Np, d), jnp.float32),
            pltpu.SemaphoreType.DMA,
            pltpu.SemaphoreType.DMA,
            pltpu.SemaphoreType.DMA,
            pltpu.SemaphoreType.DMA,
            pltpu.SemaphoreType.DMA,
            pltpu.SemaphoreType.DMA,
            pltpu.SemaphoreType.DMA,
            pltpu.SemaphoreType.DMA,
        ],
    )
    def spmm(xs, rcp, wp, out, rc_v, w_v, colix, rowix,
             g0, g1, g2, g3, acc, gm0, gm1, gm2, gm3, sm0, sm1, sm2, sm3):
        cid = lax.axis_index("c")
        sid = lax.axis_index("s")
        wid = cid * NS + sid
        pltpu.async_copy(rcp.at[wid], rc_v, gm0)
        pltpu.async_copy(wp.at[wid], w_v, gm1)

        gbufs = (g0, g1, g2, g3)
        gsems = (gm0, gm1, gm2, gm3)
        ssems = (sm0, sm1, sm2, sm3)
        zeros16 = jnp.zeros((16,), jnp.float32)

        def zrow(i, _):
            for k in range(nk):
                g0[i, pl.ds(k * 16, 16)] = zeros16
            return 0

        lax.fori_loop(0, BC, zrow, 0)
        r0 = sid * rpt

        def zacc(j, _):
            pltpu.sync_copy(g0.at[pl.ds(0, BC)],
                            acc.at[pl.ds(r0 + j * BC, BC)])
            return 0

        lax.fori_loop(0, nb, zacc, 0)
        pltpu.make_async_copy(rcp.at[wid], rc_v, gm0).wait()
        pltpu.make_async_copy(wp.at[wid], w_v, gm1).wait()
        plsc.subcore_barrier()

        def unpack_col(ch, b):
            q = ch // 4
            r = (ch % 4) * C

            def ug(g, _):
                v = rc_v[q, pl.ds(r + g * 16, 16)]
                colix[b, pl.ds(g * 16, 16)] = lax.bitwise_and(v, 16383)
                return 0

            lax.fori_loop(0, C // 16, ug, 0)

        def unpack_row(ch, b):
            q = ch // 4
            r = (ch % 4) * C

            def ug(g, _):
                v = rc_v[q, pl.ds(r + g * 16, 16)]
                rowix[b, pl.ds(g * 16, 16)] = lax.shift_right_logical(v, 14)
                return 0

            lax.fori_loop(0, C // 16, ug, 0)

        def start_gather(ch, b):
            pltpu.async_copy(xs.at[colix.at[b, pl.ds(0, 16)]],
                             gbufs[b].at[pl.ds(0, 16)], gsems[b])
            pltpu.async_copy(xs.at[colix.at[b, pl.ds(16, 16)]],
                             gbufs[b].at[pl.ds(16, 16)], gsems[b])

        def wait_gather(b):
            pltpu.make_async_copy(xs.at[colix.at[b, pl.ds(0, 16)]],
                                  gbufs[b].at[pl.ds(0, 16)],
                                  gsems[b]).wait()
            pltpu.make_async_copy(xs.at[colix.at[b, pl.ds(16, 16)]],
                                  gbufs[b].at[pl.ds(16, 16)],
                                  gsems[b]).wait()

        def start_scatter(b):
            pltpu.async_copy(gbufs[b], acc.at[rowix.at[b]], ssems[b],
                             add=True)

        def wait_scatter(b):
            pltpu.make_async_copy(gbufs[b], acc.at[rowix.at[b]],
                                  ssems[b]).wait()

        def scale(ch, b):
            gb = gbufs[b]
            q = ch // 4
            r = (ch % 4) * C

            def egroup(g, _):
                wv = w_v[q, pl.ds(r + g * 16, 16)]
                for j in range(16):
                    wb = jnp.broadcast_to(wv[j], (16,))
                    e = g * 16 + j
                    for k in range(nk):
                        sl = pl.ds(k * 16, 16)
                        gb[e, sl] = gb[e, sl] * wb
                return 0

            lax.fori_loop(0, C // 16, egroup, 0)

        nch = jnp.where(cid == 0, CH0, CH1)

        for b in range(3):
            unpack_col(b, b)
            start_gather(b, b)
        for ch in range(4):
            b = ch
            wait_gather(b)
            unpack_row(ch, b)
            scale(ch, b)
            start_scatter(b)
            nb2 = (b + 3) % 4
            if ch > 0:
                wait_scatter(nb2)
            unpack_col(ch + 3, nb2)
            start_gather(ch + 3, nb2)

        def quad(g, _):
            for boff in range(4):
                ch = 4 * g + boff
                b = boff
                wait_gather(b)
                unpack_row(ch, b)
                scale(ch, b)
                start_scatter(b)
                nb2 = (b + 3) % 4

                @pl.when(ch + 3 < nch)
                def _():
                    wait_scatter(nb2)
                    unpack_col(ch + 3, nb2)
                    start_gather(ch + 3, nb2)
            return 0

        lax.fori_loop(1, nch // 4, quad, 0)
        for b in range(4):
            wait_scatter(b)
        plsc.subcore_barrier()

        def wout(j, _):
            pltpu.sync_copy(acc.at[pl.ds(r0 + j * BC, BC)],
                            g0.at[pl.ds(0, BC)])
            pltpu.sync_copy(g0.at[pl.ds(0, BC)],
                            out.at[cid, pl.ds(r0 + j * BC, BC)])
            return 0

        lax.fori_loop(0, nb, wout, 0)

    return spmm


def _make_deg(Np, NC, NS, CH, C):
    mesh = plsc.VectorSubcoreMesh(core_axis_name="c", subcore_axis_name="s")
    rpt = Np // NS

    @functools.partial(
        pl.kernel,
        out_type=jax.ShapeDtypeStruct((NC, Np), jnp.float32),
        mesh=mesh,
        scratch_types=[
            pltpu.VMEM((CH, C), jnp.int32),
            pltpu.VMEM((CH, C), jnp.float32),
            pltpu.VMEM((rpt,), jnp.float32),
            pltpu.VMEM_SHARED((Np,), jnp.float32),
            pltpu.SemaphoreType.DMA,
            pltpu.SemaphoreType.DMA,
            pltpu.SemaphoreType.DMA,
            pltpu.SemaphoreType.DMA,
        ],
    )
    def deg(colp, wp, out, col_v, w_v, dbuf, acc, dm0, dm1, dm2, dm3):
        cid = lax.axis_index("c")
        sid = lax.axis_index("s")
        wid = cid * NS + sid
        pltpu.sync_copy(colp.at[wid], col_v)
        pltpu.sync_copy(wp.at[wid], w_v)

        zeros16 = jnp.zeros((16,), jnp.float32)

        def zrow(i, _):
            dbuf[pl.ds(i * 16, 16)] = zeros16
            return 0

        lax.fori_loop(0, rpt // 16, zrow, 0)
        r0 = sid * rpt
        pltpu.sync_copy(dbuf, acc.at[pl.ds(r0, rpt)])
        plsc.subcore_barrier()

        dsems = (dm0, dm1, dm2, dm3)

        def dstart(ch, b):
            pltpu.async_copy(w_v.at[ch], acc.at[col_v.at[ch]], dsems[b],
                             add=True)

        def dwait(b):
            pltpu.make_async_copy(w_v.at[0], acc.at[col_v.at[0]],
                                  dsems[b]).wait()

        for ch in range(4):
            dstart(ch, ch)

        def quad(g, _):
            for b in range(4):
                ch = 4 * g + b
                dwait(b)
                dstart(ch, b)
            return 0

        lax.fori_loop(1, CH // 4, quad, 0)
        for b in range(4):
            dwait(b)
        plsc.subcore_barrier()
        pltpu.sync_copy(acc.at[pl.ds(r0, rpt)], dbuf)
        pltpu.sync_copy(dbuf, out.at[cid, pl.ds(r0, rpt)])

    return deg


def _prep_body(d0_ref, d1_ref, dg_ref, dinv_ref):
    deg = d0_ref[...] + d1_ref[...]
    dg_ref[...] = lax.rsqrt(deg + 1.0)
    dinv_ref[...] = jnp.where(deg > 0, 1.0 / deg, 0.0)


def _tc_prep(d0, d1):
    N = d0.shape[0]
    return pl.pallas_call(
        _prep_body,
        out_shape=(jax.ShapeDtypeStruct((N, 1), jnp.float32),
                   jax.ShapeDtypeStruct((N, 1), jnp.float32)),
    )(d0, d1)


def _scale_body(f_ref, s_ref, o_ref):
    o_ref[...] = f_ref[...] * s_ref[...]


def _tc_scale(f, s, BN):
    N, d = f.shape
    return pl.pallas_call(
        _scale_body,
        grid=(N // BN,),
        in_specs=[pl.BlockSpec((BN, d), lambda i: (i, 0)),
                  pl.BlockSpec((BN, 1), lambda i: (i, 0))],
        out_specs=pl.BlockSpec((BN, d), lambda i: (i, 0)),
        out_shape=jax.ShapeDtypeStruct((N, d), jnp.float32),
    )(f, s)


def _gcn_body(p0_ref, p1_ref, g_ref, dg_ref, gn_ref, br_ref):
    fn = dg_ref[...] * (p0_ref[...] + p1_ref[...] + g_ref[...])
    gn_ref[...] = dg_ref[...] * fn
    br_ref[...] = _leaky(fn)


def _tc_combine_gcn(p0, p1, g, dg, BN):
    N, d = g.shape
    return pl.pallas_call(
        _gcn_body,
        grid=(N // BN,),
        in_specs=[pl.BlockSpec((BN, d), lambda i: (i, 0)),
                  pl.BlockSpec((BN, d), lambda i: (i, 0)),
                  pl.BlockSpec((BN, d), lambda i: (i, 0)),
                  pl.BlockSpec((BN, 1), lambda i: (i, 0))],
        out_specs=(pl.BlockSpec((BN, d), lambda i: (i, 0)),
                   pl.BlockSpec((BN, d), lambda i: (i, 0))),
        out_shape=(jax.ShapeDtypeStruct((N, d), jnp.float32),
                   jax.ShapeDtypeStruct((N, d), jnp.float32)),
    )(p0, p1, g, dg)


def _sct_body(p0_ref, p1_ref, fp_ref, dinv_ref, m_ref, fn_ref, dx_ref, br_ref):
    fn = 0.5 * fp_ref[...] + 0.5 * (p0_ref[...] + p1_ref[...])
    fn_ref[...] = fn
    dx_ref[...] = dinv_ref[...] * fn
    ad = jnp.abs(fp_ref[...] - fn)
    m = m_ref[...]
    br_ref[...] = jnp.where(
        ad > 0, jnp.exp(m * jnp.log(jnp.maximum(ad, 1e-38))), 0.0)


def _tc_combine_sct(p0, p1, fp, dinv, m, BN):
    N, d = fp.shape
    return pl.pallas_call(
        _sct_body,
        grid=(N // BN,),
        in_specs=[pl.BlockSpec((BN, d), lambda i: (i, 0)),
                  pl.BlockSpec((BN, d), lambda i: (i, 0)),
                  pl.BlockSpec((BN, d), lambda i: (i, 0)),
                  pl.BlockSpec((BN, 1), lambda i: (i, 0)),
                  pl.BlockSpec((1, 1), lambda i: (0, 0))],
        out_specs=(pl.BlockSpec((BN, d), lambda i: (i, 0)),
                   pl.BlockSpec((BN, d), lambda i: (i, 0)),
                   pl.BlockSpec((BN, d), lambda i: (i, 0))),
        out_shape=(jax.ShapeDtypeStruct((N, d), jnp.float32),
                   jax.ShapeDtypeStruct((N, d), jnp.float32),
                   jax.ShapeDtypeStruct((N, d), jnp.float32)),
    )(p0, p1, fp, dinv, m)


def _attend_body(x_ref, h0, h1, h2, h3, h4, h5, a_ref, w1_ref, b1_ref,
                 w2_ref, b2_ref, o_ref):
    d = x_ref.shape[1]
    a1 = a_ref[pl.ds(0, d), :]
    a2 = a_ref[pl.ds(d, d), :]
    c = jnp.dot(jnp.maximum(x_ref[...], 0.0), a1,
                preferred_element_type=jnp.float32)
    hs = [h0[...], h1[...], h2[...], h3[...], h4[...], h5[...]]
    es = [c + jnp.dot(jnp.maximum(h, 0.0), a2,
                      preferred_element_type=jnp.float32) for h in hs]
    e = jnp.concatenate(es, axis=1)
    e = e - jnp.max(e, axis=1, keepdims=True)
    ex = jnp.exp(e)
    att = ex / jnp.sum(ex, axis=1, keepdims=True)
    hp = att[:, 0:1] * hs[0]
    for k in range(1, 6):
        hp = hp + att[:, k:k + 1] * hs[k]
    hp = hp * (1.0 / 6.0)
    t = _leaky(lax.dot_general(hp, w1_ref[...], (((1,), (1,)), ((), ())),
                               preferred_element_type=jnp.float32)
               + b1_ref[...])
    o_ref[...] = _leaky(
        lax.dot_general(t, w2_ref[...], (((1,), (1,)), ((), ())),
                        preferred_element_type=jnp.float32) + b2_ref[...])


def _tc_attend(x, hs, a, w1, b1, w2, b2, BN):
    N, d = x.shape
    blk = pl.BlockSpec((BN, d), lambda i: (i, 0))
    return pl.pallas_call(
        _attend_body,
        grid=(N // BN,),
        in_specs=[blk, blk, blk, blk, blk, blk, blk,
                  pl.BlockSpec((2 * d, 1), lambda i: (0, 0)),
                  pl.BlockSpec((d, d), lambda i: (0, 0)),
                  pl.BlockSpec((1, d), lambda i: (0, 0)),
                  pl.BlockSpec((d, d), lambda i: (0, 0)),
                  pl.BlockSpec((1, d), lambda i: (0, 0))],
        out_specs=blk,
        out_shape=jax.ShapeDtypeStruct((N, d), jnp.float32),
    )(x, *hs, a, w1, b1, w2, b2)


def kernel(X, edge_index, edge_weight, W1, b1, W2, b2, a, moment):
    N, d = X.shape
    E = edge_weight.shape[0]
    NC, NS, _ = _sc_geometry()
    T = NC * NS
    C = 32
    tot = max(8, 4 * (-(-E // (NS * C * 4))))
    FR0 = 0.608
    CH0 = max(4, 4 * round(tot * FR0 / 4))
    CH1 = tot - CH0
    CHm = max(CH0, CH1)
    pad = NS * tot * C - E
    rowf = jnp.pad(edge_index[0], (0, pad))
    colf = jnp.pad(edge_index[1], (0, pad))
    wf = jnp.pad(edge_weight, (0, pad))
    rcf = jnp.bitwise_or(jnp.left_shift(rowf, 14), colf)

    def _slab(x):
        p0 = x[:NS * CH0 * C].reshape(NS, CH0, C)
        p1 = x[NS * CH0 * C:].reshape(NS, CH1, C)
        p0 = jnp.pad(p0, ((0, 0), (0, CHm - CH0), (0, 0)))
        p1 = jnp.pad(p1, ((0, 0), (0, CHm - CH1), (0, 0)))
        return jnp.concatenate([p0, p1], axis=0)

    rc = _slab(rcf).reshape(T, CHm * C // 128, 128)
    w_s = _slab(wf).reshape(T, CHm * C // 128, 128)

    CHd = max(4, 4 * (-(-E // (T * C * 4))))
    padd = T * CHd * C - E
    col = jnp.pad(edge_index[1], (0, padd)).reshape(T, CHd, C)
    w = jnp.pad(edge_weight, (0, padd)).reshape(T, CHd, C)

    rpt = -(-N // NS)
    Np = NS * (-(-rpt // 128) * 128)
    degp = _make_deg(Np, NC, NS, CHd, C)(col, w)
    dg, dinv = _tc_prep(degp[0, :N, None], degp[1, :N, None])

    BN = 1000 if N % 1000 == 0 else 8
    spmm = _make_spmm(Np, d, NC, NS, CHm, C, CH0, CH1)

    branches = []
    g = _tc_scale(X, dg, BN)
    for _ in range(3):
        p = spmm(g, rc, w_s)
        g, br = _tc_combine_gcn(p[0, :N], p[1, :N], g, dg, BN)
        branches.append(br)

    m = jnp.asarray(moment, jnp.float32).reshape(1, 1)
    fp = X
    dix = _tc_scale(X, dinv, BN)
    for t in range(4):
        p = spmm(dix, rc, w_s)
        fp, dix, br = _tc_combine_sct(p[0, :N], p[1, :N], fp, dinv, m, BN)
        if t > 0:
            branches.append(br)

    b1r = b1.reshape(1, d)
    b2r = b2.reshape(1, d)
    return _tc_attend(X, branches, a, W1, b1r, W2, b2r, BN)
